# Initial kernel scaffold; baseline (speedup 1.0000x reference)
import jax
import jax.numpy as jnp
from jax.experimental import pallas as pl


def kernel(x, edge_index, W, b):
    raise NotImplementedError("write your pallas kernel here")


# speedup vs baseline: 11.7403x; 11.7403x over previous
import functools

import jax
import jax.numpy as jnp
from jax import lax
from jax.experimental import pallas as pl
from jax.experimental.pallas import tpu as pltpu
from jax.experimental.pallas import tpu_sc as plsc

N = 10000
E = 160000
D = 256
HD = 128
NC = 2
NS = 16
NP = 10112
DUMMY = N

CH1 = 40
E1 = NC * NS * CH1 * 128
CH3 = 79
E3 = NS * CH3 * 128

ROWS_PER_TILE = NP // NS


def _hist_body(dst_hbm, e0_hbm, zbuf_hbm, hist_out, dst_v, e0_v, buf_v, hist_sp):
    c = lax.axis_index("c")
    s = lax.axis_index("s")
    pltpu.sync_copy(dst_hbm.at[c, s], dst_v)
    pltpu.sync_copy(e0_hbm, e0_v)
    pltpu.sync_copy(zbuf_hbm, buf_v)
    base = s * ROWS_PER_TILE
    for k in range(4):
        pltpu.sync_copy(buf_v, hist_sp.at[pl.ds(base + k * 128, 128)])
    rem = ROWS_PER_TILE - 4 * 128
    pltpu.sync_copy(buf_v.at[pl.ds(0, rem)],
                    hist_sp.at[pl.ds(base + 4 * 128, rem)])
    plsc.subcore_barrier()

    def chunk(j, carry):
        pltpu.sync_copy(e0_v, hist_sp.at[dst_v.at[j]], add=True)
        return carry

    lax.fori_loop(0, CH1, chunk, 0)
    plsc.subcore_barrier()
    for k in range(4):
        pltpu.sync_copy(hist_sp.at[pl.ds(base + k * 128, 128)], buf_v)
        pltpu.sync_copy(buf_v, hist_out.at[c, pl.ds(base + k * 128, 128)])
    pltpu.sync_copy(hist_sp.at[pl.ds(base + 4 * 128, rem)],
                    buf_v.at[pl.ds(0, rem)])
    pltpu.sync_copy(buf_v.at[pl.ds(0, rem)],
                    hist_out.at[c, pl.ds(base + 4 * 128, rem)])


_hist_kernel = pl.kernel(
    _hist_body,
    out_type=jax.ShapeDtypeStruct((NC, NP, 128), jnp.float32),
    mesh=plsc.VectorSubcoreMesh(
        core_axis_name="c", subcore_axis_name="s", num_cores=NC, num_subcores=NS
    ),
    scratch_types=[
        pltpu.VMEM((CH1, 128), jnp.int32),
        pltpu.VMEM((128, 128), jnp.float32),
        pltpu.VMEM((128, 128), jnp.float32),
        pltpu.VMEM_SHARED((NP, 128), jnp.float32),
    ],
)


def _scatter_body(g_hbm, src_hbm, dst_hbm, zbuf_hbm, acc_out,
                  src_v, dst_v, buf_v, acc_sp):
    c = lax.axis_index("c")
    s = lax.axis_index("s")
    pltpu.sync_copy(src_hbm.at[c, s], src_v)
    pltpu.sync_copy(dst_hbm.at[s], dst_v)
    pltpu.sync_copy(zbuf_hbm, buf_v)
    base = s * ROWS_PER_TILE
    for k in range(4):
        pltpu.sync_copy(buf_v, acc_sp.at[pl.ds(base + k * 128, 128)])
    rem = ROWS_PER_TILE - 4 * 128
    pltpu.sync_copy(buf_v.at[pl.ds(0, rem)],
                    acc_sp.at[pl.ds(base + 4 * 128, rem)])
    plsc.subcore_barrier()

    def chunk(j, carry):
        pltpu.sync_copy(g_hbm.at[src_v.at[j]], buf_v)
        pltpu.sync_copy(buf_v, acc_sp.at[dst_v.at[j]], add=True)
        return carry

    lax.fori_loop(0, CH3, chunk, 0)
    plsc.subcore_barrier()
    for k in range(4):
        pltpu.sync_copy(acc_sp.at[pl.ds(base + k * 128, 128)], buf_v)
        pltpu.sync_copy(buf_v, acc_out.at[c, pl.ds(base + k * 128, 128)])
    pltpu.sync_copy(acc_sp.at[pl.ds(base + 4 * 128, rem)],
                    buf_v.at[pl.ds(0, rem)])
    pltpu.sync_copy(buf_v.at[pl.ds(0, rem)],
                    acc_out.at[c, pl.ds(base + 4 * 128, rem)])


_scatter_kernel = pl.kernel(
    _scatter_body,
    out_type=jax.ShapeDtypeStruct((NC, NP, HD), jnp.float32),
    mesh=plsc.VectorSubcoreMesh(
        core_axis_name="c", subcore_axis_name="s", num_cores=NC, num_subcores=NS
    ),
    scratch_types=[
        pltpu.VMEM((CH3, 128), jnp.int32),
        pltpu.VMEM((CH3, 128), jnp.int32),
        pltpu.VMEM((128, HD), jnp.float32),
        pltpu.VMEM_SHARED((NP, HD), jnp.float32),
    ],
)


BM = 1000


def _matmul_body(x_ref, w_ref, hist_ref, g_ref):
    deg = hist_ref[0, :, 0] + hist_ref[1, :, 0] + 1.0
    dis = lax.rsqrt(deg)
    h = jnp.dot(x_ref[...], w_ref[...], preferred_element_type=jnp.float32)
    h = h * dis[:, None]
    g_ref[0] = h[:, :HD]
    g_ref[1] = h[:, HD:]


def _matmul(x, w, hist):
    return pl.pallas_call(
        _matmul_body,
        grid=(N // BM,),
        in_specs=[
            pl.BlockSpec((BM, D), lambda i: (i, 0)),
            pl.BlockSpec((D, D), lambda i: (0, 0)),
            pl.BlockSpec((NC, BM, 128), lambda i: (0, i, 0)),
        ],
        out_specs=pl.BlockSpec((NC, BM, HD), lambda i: (0, i, 0)),
        out_shape=jax.ShapeDtypeStruct((NC, N, HD), jnp.float32),
    )(x, w, hist)


def _finish_body(acc_ref, g_ref, hist_ref, b_ref, out_ref):
    deg = hist_ref[0, :, 0] + hist_ref[1, :, 0] + 1.0
    dis = lax.rsqrt(deg)
    lo = acc_ref[0] + g_ref[0]
    hi = acc_ref[1] + g_ref[1]
    out_ref[...] = jnp.concatenate([lo, hi], axis=1) * dis[:, None] + b_ref[...]


def _finish(acc, g, hist, b):
    return pl.pallas_call(
        _finish_body,
        grid=(N // BM,),
        in_specs=[
            pl.BlockSpec((NC, BM, HD), lambda i: (0, i, 0)),
            pl.BlockSpec((NC, BM, HD), lambda i: (0, i, 0)),
            pl.BlockSpec((NC, BM, 128), lambda i: (0, i, 0)),
            pl.BlockSpec((1, D), lambda i: (0, 0)),
        ],
        out_specs=pl.BlockSpec((BM, D), lambda i: (i, 0)),
        out_shape=jax.ShapeDtypeStruct((N, D), jnp.float32),
    )(acc, g, hist, b)


@jax.jit
def kernel(x, edge_index, W, b):
    src = edge_index[0].astype(jnp.int32)
    dst = edge_index[1].astype(jnp.int32)

    dst1 = jnp.concatenate(
        [dst, jnp.full((E1 - E,), DUMMY, jnp.int32)]).reshape(NC, NS, CH1, 128)
    e0 = jnp.zeros((128, 128), jnp.float32).at[:, 0].set(1.0)
    zbuf1 = jnp.zeros((128, 128), jnp.float32)
    hist = _hist_kernel(dst1, e0, zbuf1)

    g3 = _matmul(x, W, hist)

    pad3 = jnp.zeros((E3 - E,), jnp.int32)
    src3 = jnp.concatenate([src, pad3]).reshape(NS, CH3, 128)
    srcT = src3[None] + (jnp.arange(NC, dtype=jnp.int32) * N)[:, None, None, None]
    dst3 = jnp.concatenate(
        [dst, jnp.full((E3 - E,), DUMMY, jnp.int32)]).reshape(NS, CH3, 128)
    zbuf = jnp.zeros((128, HD), jnp.float32)
    acc = _scatter_kernel(g3.reshape(NC * N, HD), srcT, dst3, zbuf)

    return _finish(acc, g3, hist, b.reshape(1, D))
